# baseline (device time: 56097 ns/iter reference)
import jax
import jax.numpy as jnp
from jax import lax
from jax.experimental import pallas as pl
from jax.experimental.pallas import tpu as pltpu

N_DEV = 16
HOPS = 8
SUB = 8


def _gelu_f32(y):
    c = 0.7978845608028654
    return 0.5 * y * (1.0 + jnp.tanh(c * (y + 0.044715 * y * y * y)))


def kernel(x, w_mat):
    m_per, k = x.shape
    _, n_per = w_mat.shape
    msub = m_per // SUB

    def _r_active(s, j):
        return s < HOPS - 1 or j < SUB // 2

    def _l_active(s, j):
        return s < HOPS - 1 or j >= SUB // 2

    def _r_dst(s):
        return s + 1

    def _l_dst(s):
        return 8 if s == HOPS - 1 else 9 + s

    def _r_src(s):
        return 0 if s == 0 else s

    def _l_src(s):
        return 0 if s == 0 else 8 + s

    def body(x_ref, w_ref, out_ref, comm_ref, w_bf, send_r, recv_r,
             send_l, recv_l):
        my = lax.axis_index("i")
        left = (my - 1) % N_DEV
        right = (my + 1) % N_DEV

        barrier_sem = pltpu.get_barrier_semaphore()
        for nbr in (left, right):
            pl.semaphore_signal(
                barrier_sem, inc=1,
                device_id=(nbr,), device_id_type=pl.DeviceIdType.MESH,
            )
        pl.semaphore_wait(barrier_sem, 2)

        comm_ref[0, :, :] = x_ref[:, :].astype(jnp.bfloat16)

        sends = []

        def _send(src_slot, dst_slot, j, ssem, rsem, dst_dev):
            rows = pl.ds(j * msub, msub)
            rdma = pltpu.make_async_remote_copy(
                src_ref=comm_ref.at[src_slot, rows],
                dst_ref=comm_ref.at[dst_slot, rows],
                send_sem=ssem, recv_sem=rsem,
                device_id=(dst_dev,),
                device_id_type=pl.DeviceIdType.MESH,
            )
            rdma.start()
            sends.append(rdma)

        def _wait_recv(dst_slot, j, ssem, rsem):
            rows = pl.ds(j * msub, msub)
            rdma = pltpu.make_async_remote_copy(
                src_ref=comm_ref.at[dst_slot, rows],
                dst_ref=comm_ref.at[dst_slot, rows],
                send_sem=ssem, recv_sem=rsem,
                device_id=(left,), device_id_type=pl.DeviceIdType.MESH,
            )
            rdma.wait_recv()

        for j in range(SUB):
            _send(0, _r_dst(0), j, send_r.at[0, j], recv_r.at[0, j], right)
            _send(0, _l_dst(0), j, send_l.at[0, j], recv_l.at[0, j], left)

        w_bf[:, :] = w_ref[:, :].astype(jnp.bfloat16)
        y0 = jnp.dot(comm_ref[0, :, :], w_bf[:, :],
                     preferred_element_type=jnp.float32)
        out_ref[pl.ds(my * m_per, m_per), :] = _gelu_f32(y0)

        for s in range(HOPS):
            for j in range(SUB):
                if _r_active(s, j):
                    _wait_recv(_r_dst(s), j, send_r.at[s, j], recv_r.at[s, j])
                    if s + 1 < HOPS and _r_active(s + 1, j):
                        _send(_r_src(s + 1), _r_dst(s + 1), j,
                              send_r.at[s + 1, j], recv_r.at[s + 1, j], right)
                if _l_active(s, j):
                    _wait_recv(_l_dst(s), j, send_l.at[s, j], recv_l.at[s, j])
                    if s + 1 < HOPS and _l_active(s + 1, j):
                        _send(_l_src(s + 1), _l_dst(s + 1), j,
                              send_l.at[s + 1, j], recv_l.at[s + 1, j], left)

            if s < HOPS - 1:
                origin_r = (my - s - 1) % N_DEV
                yr = jnp.dot(comm_ref[s + 1, :, :], w_bf[:, :],
                             preferred_element_type=jnp.float32)
                out_ref[pl.ds(origin_r * m_per, m_per), :] = _gelu_f32(yr)
                origin_l = (my + s + 1) % N_DEV
                yl = jnp.dot(comm_ref[9 + s, :, :], w_bf[:, :],
                             preferred_element_type=jnp.float32)
                out_ref[pl.ds(origin_l * m_per, m_per), :] = _gelu_f32(yl)
            else:
                origin_a = (my + HOPS) % N_DEV
                ya = jnp.dot(comm_ref[8, :, :], w_bf[:, :],
                             preferred_element_type=jnp.float32)
                out_ref[pl.ds(origin_a * m_per, m_per), :] = _gelu_f32(ya)

        for rdma in sends:
            rdma.wait_send()

    return pl.pallas_call(
        body,
        out_shape=jax.ShapeDtypeStruct((N_DEV * m_per, n_per), jnp.float32),
        in_specs=[
            pl.BlockSpec(memory_space=pltpu.VMEM),
            pl.BlockSpec(memory_space=pltpu.VMEM),
        ],
        out_specs=pl.BlockSpec(memory_space=pltpu.VMEM),
        scratch_shapes=[
            pltpu.VMEM((N_DEV, m_per, k), jnp.bfloat16),
            pltpu.VMEM((k, n_per), jnp.bfloat16),
            pltpu.SemaphoreType.DMA((HOPS, SUB)),
            pltpu.SemaphoreType.DMA((HOPS, SUB)),
            pltpu.SemaphoreType.DMA((HOPS, SUB)),
            pltpu.SemaphoreType.DMA((HOPS, SUB)),
        ],
        compiler_params=pltpu.CompilerParams(collective_id=0),
    )(x, w_mat)


# device time: 54875 ns/iter; 1.0223x vs baseline; 1.0223x over previous
import jax
import jax.numpy as jnp
from jax import lax
from jax.experimental import pallas as pl
from jax.experimental.pallas import tpu as pltpu

N_DEV = 16
HOPS = 8
SUB = 4


def _gelu_f32(y):
    c = 0.7978845608028654
    return 0.5 * y * (1.0 + jnp.tanh(c * (y + 0.044715 * y * y * y)))


def kernel(x, w_mat):
    m_per, k = x.shape
    _, n_per = w_mat.shape
    msub = m_per // SUB

    def _r_active(s, j):
        return s < HOPS - 1 or j < SUB // 2

    def _l_active(s, j):
        return s < HOPS - 1 or j >= SUB // 2

    def _r_dst(s):
        return s + 1

    def _l_dst(s):
        return 8 if s == HOPS - 1 else 9 + s

    def _r_src(s):
        return 0 if s == 0 else s

    def _l_src(s):
        return 0 if s == 0 else 8 + s

    def body(x_ref, w_ref, out_ref, comm_ref, w_bf, send_r, recv_r,
             send_l, recv_l):
        my = lax.axis_index("i")
        left = (my - 1) % N_DEV
        right = (my + 1) % N_DEV

        barrier_sem = pltpu.get_barrier_semaphore()
        for nbr in (left, right):
            pl.semaphore_signal(
                barrier_sem, inc=1,
                device_id=(nbr,), device_id_type=pl.DeviceIdType.MESH,
            )
        pl.semaphore_wait(barrier_sem, 2)

        comm_ref[0, :, :] = x_ref[:, :].astype(jnp.bfloat16)

        sends = []

        def _send(src_slot, dst_slot, j, ssem, rsem, dst_dev):
            rows = pl.ds(j * msub, msub)
            rdma = pltpu.make_async_remote_copy(
                src_ref=comm_ref.at[src_slot, rows],
                dst_ref=comm_ref.at[dst_slot, rows],
                send_sem=ssem, recv_sem=rsem,
                device_id=(dst_dev,),
                device_id_type=pl.DeviceIdType.MESH,
            )
            rdma.start()
            sends.append(rdma)

        def _wait_recv(dst_slot, j, ssem, rsem):
            rows = pl.ds(j * msub, msub)
            rdma = pltpu.make_async_remote_copy(
                src_ref=comm_ref.at[dst_slot, rows],
                dst_ref=comm_ref.at[dst_slot, rows],
                send_sem=ssem, recv_sem=rsem,
                device_id=(left,), device_id_type=pl.DeviceIdType.MESH,
            )
            rdma.wait_recv()

        for j in range(SUB):
            _send(0, _r_dst(0), j, send_r.at[0, j], recv_r.at[0, j], right)
            _send(0, _l_dst(0), j, send_l.at[0, j], recv_l.at[0, j], left)

        w_bf[:, :] = w_ref[:, :].astype(jnp.bfloat16)
        y0 = jnp.dot(comm_ref[0, :, :], w_bf[:, :],
                     preferred_element_type=jnp.float32)
        out_ref[pl.ds(my * m_per, m_per), :] = _gelu_f32(y0)

        for s in range(HOPS):
            for j in range(SUB):
                if _r_active(s, j):
                    _wait_recv(_r_dst(s), j, send_r.at[s, j], recv_r.at[s, j])
                    if s + 1 < HOPS and _r_active(s + 1, j):
                        _send(_r_src(s + 1), _r_dst(s + 1), j,
                              send_r.at[s + 1, j], recv_r.at[s + 1, j], right)
                if _l_active(s, j):
                    _wait_recv(_l_dst(s), j, send_l.at[s, j], recv_l.at[s, j])
                    if s + 1 < HOPS and _l_active(s + 1, j):
                        _send(_l_src(s + 1), _l_dst(s + 1), j,
                              send_l.at[s + 1, j], recv_l.at[s + 1, j], left)

            if s < HOPS - 1:
                origin_r = (my - s - 1) % N_DEV
                yr = jnp.dot(comm_ref[s + 1, :, :], w_bf[:, :],
                             preferred_element_type=jnp.float32)
                out_ref[pl.ds(origin_r * m_per, m_per), :] = _gelu_f32(yr)
                origin_l = (my + s + 1) % N_DEV
                yl = jnp.dot(comm_ref[9 + s, :, :], w_bf[:, :],
                             preferred_element_type=jnp.float32)
                out_ref[pl.ds(origin_l * m_per, m_per), :] = _gelu_f32(yl)
            else:
                origin_a = (my + HOPS) % N_DEV
                ya = jnp.dot(comm_ref[8, :, :], w_bf[:, :],
                             preferred_element_type=jnp.float32)
                out_ref[pl.ds(origin_a * m_per, m_per), :] = _gelu_f32(ya)

        for rdma in sends:
            rdma.wait_send()

    return pl.pallas_call(
        body,
        out_shape=jax.ShapeDtypeStruct((N_DEV * m_per, n_per), jnp.float32),
        in_specs=[
            pl.BlockSpec(memory_space=pltpu.VMEM),
            pl.BlockSpec(memory_space=pltpu.VMEM),
        ],
        out_specs=pl.BlockSpec(memory_space=pltpu.VMEM),
        scratch_shapes=[
            pltpu.VMEM((N_DEV, m_per, k), jnp.bfloat16),
            pltpu.VMEM((k, n_per), jnp.bfloat16),
            pltpu.SemaphoreType.DMA((HOPS, SUB)),
            pltpu.SemaphoreType.DMA((HOPS, SUB)),
            pltpu.SemaphoreType.DMA((HOPS, SUB)),
            pltpu.SemaphoreType.DMA((HOPS, SUB)),
        ],
        compiler_params=pltpu.CompilerParams(collective_id=0),
    )(x, w_mat)


# device time: 45263 ns/iter; 1.2394x vs baseline; 1.2124x over previous
import jax
import jax.numpy as jnp
from jax import lax
from jax.experimental import pallas as pl
from jax.experimental.pallas import tpu as pltpu

N_DEV = 16

N_C = 7


def _cidx(kind, d):
    return d if kind == "bel" else 3 + d


def _gelu_f32(y):
    c = 0.7978845608028654
    return 0.5 * y * (1.0 + jnp.tanh(c * (y + 0.044715 * y * y * y)))


def kernel(x, w_mat):
    m_per, k = x.shape
    _, n_per = w_mat.shape
    mh = m_per // 2

    def body(x_ref, w_ref, out_ref, comm_ref, w_bf,
             csend, crecv, ssend, srecv):
        my = lax.axis_index("i")
        z = my // 4
        q = my % 4
        up = my + 4
        down = my - 4
        right = 4 * z + (q + 1) % 4
        left = 4 * z + (q + 3) % 4

        has_up = z < 3
        has_dn = z > 0

        def pred_c(c):
            if c == 0:
                return None
            if c <= 3:
                return z >= c
            return z + (c - 3) <= 3

        dq = {"me": 0, "L": 3, "R": 1, "D": 2}

        def origin(kind, c):
            qq = (q + dq[kind]) % 4
            if c == 0:
                return 4 * z + qq
            if c <= 3:
                return 4 * (z - c) + qq
            return 4 * (z + (c - 3)) + qq

        barrier_sem = pltpu.get_barrier_semaphore()
        for nbr in (left, right):
            pl.semaphore_signal(barrier_sem, inc=1, device_id=(nbr,),
                                device_id_type=pl.DeviceIdType.MESH)

        @pl.when(has_up)
        def _():
            pl.semaphore_signal(barrier_sem, inc=1, device_id=(up,),
                                device_id_type=pl.DeviceIdType.MESH)

        @pl.when(has_dn)
        def _():
            pl.semaphore_signal(barrier_sem, inc=1, device_id=(down,),
                                device_id_type=pl.DeviceIdType.MESH)

        pl.semaphore_wait(barrier_sem, 2)

        @pl.when(has_up)
        def _():
            pl.semaphore_wait(barrier_sem, 1)

        @pl.when(has_dn)
        def _():
            pl.semaphore_wait(barrier_sem, 1)

        comm_ref[0, :, :] = x_ref[:, :].astype(jnp.bfloat16)

        sends = []

        def _rdma(src_slot, dst_slot, dev, ssem, rsem, rows=None):
            sl = slice(None) if rows is None else rows
            return pltpu.make_async_remote_copy(
                src_ref=comm_ref.at[src_slot, sl],
                dst_ref=comm_ref.at[dst_slot, sl],
                send_sem=ssem, recv_sem=rsem,
                device_id=(dev,), device_id_type=pl.DeviceIdType.MESH,
            )

        def _start(rdma, pred):
            if pred is None:
                rdma.start()
            else:
                @pl.when(pred)
                def _():
                    rdma.start()
            sends.append((rdma, pred))

        def _wait(rdma, pred):
            if pred is None:
                rdma.wait_recv()
            else:
                @pl.when(pred)
                def _():
                    rdma.wait_recv()

        _start(_rdma(0, 1, up, csend.at[0], crecv.at[0]), has_up)
        _start(_rdma(0, 4, down, csend.at[1], crecv.at[3]), has_dn)

        _start(_rdma(0, 7 + 0, right, ssend.at[0, 0], srecv.at[0, 0]), None)
        _start(_rdma(0, 14 + 0, left, ssend.at[0, 1], srecv.at[0, 1]), None)

        w_bf[:, :] = w_ref[:, :].astype(jnp.bfloat16)
        y0 = jnp.dot(comm_ref[0, :, :], w_bf[:, :],
                     preferred_element_type=jnp.float32)
        out_ref[pl.ds(my * m_per, m_per), :] = _gelu_f32(y0)

        def col_step(d):
            for kind in ("bel", "abv"):
                c = _cidx(kind, d)
                p = pred_c(c)
                isem = c - 1 if kind == "bel" else 3 + (c - 4)
                _wait(_rdma(c, c, left, csend.at[0], crecv.at[isem]), p)
                if d < 3:
                    nc = c + 1
                    if kind == "bel":
                        fp = jnp.logical_and(p, has_up)
                        fsem = 2 + (d - 1)
                        _start(_rdma(c, nc, up, csend.at[fsem],
                                     crecv.at[nc - 1]), fp)
                    else:
                        fp = jnp.logical_and(p, has_dn)
                        fsem = 4 + (d - 1)
                        _start(_rdma(c, nc, down, csend.at[fsem],
                                     crecv.at[3 + (nc - 4)]), fp)
                _start(_rdma(c, 7 + c, right, ssend.at[c, 0],
                             srecv.at[c, 0]), p)
                _start(_rdma(c, 14 + c, left, ssend.at[c, 1],
                             srecv.at[c, 1]), p)
                if p is None:
                    yc = jnp.dot(comm_ref[c, :, :], w_bf[:, :],
                                 preferred_element_type=jnp.float32)
                    out_ref[pl.ds(origin("me", c) * m_per, m_per), :] = \
                        _gelu_f32(yc)
                else:
                    @pl.when(p)
                    def _():
                        yc = jnp.dot(comm_ref[c, :, :], w_bf[:, :],
                                     preferred_element_type=jnp.float32)
                        out_ref[pl.ds(origin("me", c) * m_per, m_per), :] = \
                            _gelu_f32(yc)

        def sq_step(c):
            p = pred_c(c)
            _wait(_rdma(7 + c, 7 + c, left, ssend.at[c, 0],
                        srecv.at[c, 0]), p)
            _start(_rdma(7 + c, 21 + c, right, ssend.at[c, 2],
                         srecv.at[c, 2], rows=pl.ds(0, mh)), p)
            _wait(_rdma(14 + c, 14 + c, left, ssend.at[c, 1],
                        srecv.at[c, 1]), p)
            _start(_rdma(14 + c, 21 + c, left, ssend.at[c, 3],
                         srecv.at[c, 3], rows=pl.ds(mh, mh)), p)

            def _gemms():
                yl = jnp.dot(comm_ref[7 + c, :, :], w_bf[:, :],
                             preferred_element_type=jnp.float32)
                out_ref[pl.ds(origin("L", c) * m_per, m_per), :] = \
                    _gelu_f32(yl)
                yr = jnp.dot(comm_ref[14 + c, :, :], w_bf[:, :],
                             preferred_element_type=jnp.float32)
                out_ref[pl.ds(origin("R", c) * m_per, m_per), :] = \
                    _gelu_f32(yr)

            if p is None:
                _gemms()
            else:
                @pl.when(p)
                def _():
                    _gemms()

        def diag_step(c):
            p = pred_c(c)
            _wait(_rdma(21 + c, 21 + c, left, ssend.at[c, 2],
                        srecv.at[c, 2], rows=pl.ds(0, mh)), p)
            _wait(_rdma(21 + c, 21 + c, left, ssend.at[c, 3],
                        srecv.at[c, 3], rows=pl.ds(mh, mh)), p)

            def _gemm():
                yd = jnp.dot(comm_ref[21 + c, :, :], w_bf[:, :],
                             preferred_element_type=jnp.float32)
                out_ref[pl.ds(origin("D", c) * m_per, m_per), :] = \
                    _gelu_f32(yd)

            if p is None:
                _gemm()
            else:
                @pl.when(p)
                def _():
                    _gemm()

        col_step(1)
        sq_step(0)
        col_step(2)
        sq_step(1)
        sq_step(4)
        col_step(3)
        sq_step(2)
        sq_step(5)
        diag_step(0)
        sq_step(3)
        sq_step(6)
        diag_step(1)
        diag_step(4)
        diag_step(2)
        diag_step(5)
        diag_step(3)
        diag_step(6)

        for rdma, pred in sends:
            if pred is None:
                rdma.wait_send()
            else:
                @pl.when(pred)
                def _():
                    rdma.wait_send()

    return pl.pallas_call(
        body,
        out_shape=jax.ShapeDtypeStruct((N_DEV * m_per, n_per), jnp.float32),
        in_specs=[
            pl.BlockSpec(memory_space=pltpu.VMEM),
            pl.BlockSpec(memory_space=pltpu.VMEM),
        ],
        out_specs=pl.BlockSpec(memory_space=pltpu.VMEM),
        scratch_shapes=[
            pltpu.VMEM((28, m_per, k), jnp.bfloat16),
            pltpu.VMEM((k, n_per), jnp.bfloat16),
            pltpu.SemaphoreType.DMA((6,)),
            pltpu.SemaphoreType.DMA((6,)),
            pltpu.SemaphoreType.DMA((N_C, 4)),
            pltpu.SemaphoreType.DMA((N_C, 4)),
        ],
        compiler_params=pltpu.CompilerParams(collective_id=0),
    )(x, w_mat)


# device time: 43358 ns/iter; 1.2938x vs baseline; 1.0439x over previous
import jax
import jax.numpy as jnp
from jax import lax
from jax.experimental import pallas as pl
from jax.experimental.pallas import tpu as pltpu

N_DEV = 16

N_C = 7


def _cidx(kind, d):
    return d if kind == "bel" else 3 + d


def _gelu_f32(y):
    c = 0.7978845608028654
    return 0.5 * y * (1.0 + jnp.tanh(c * (y + 0.044715 * y * y * y)))


def kernel(x, w_mat):
    m_per, k = x.shape
    _, n_per = w_mat.shape
    mh = m_per // 2

    def body(x_ref, w_ref, out_ref, comm_ref, w_bf,
             csend, crecv, ssend, srecv):
        my = lax.axis_index("i")
        z = my // 4
        q = my % 4
        up = my + 4
        down = my - 4
        right = 4 * z + (q + 1) % 4
        left = 4 * z + (q + 3) % 4

        has_up = z < 3
        has_dn = z > 0

        def pred_c(c):
            if c == 0:
                return None
            if c <= 3:
                return z >= c
            return z + (c - 3) <= 3

        dq = {"me": 0, "L": 3, "R": 1, "D": 2}

        def origin(kind, c):
            qq = (q + dq[kind]) % 4
            if c == 0:
                return 4 * z + qq
            if c <= 3:
                return 4 * (z - c) + qq
            return 4 * (z + (c - 3)) + qq

        barrier_sem = pltpu.get_barrier_semaphore()
        for nbr in (left, right):
            pl.semaphore_signal(barrier_sem, inc=1, device_id=(nbr,),
                                device_id_type=pl.DeviceIdType.MESH)

        @pl.when(has_up)
        def _():
            pl.semaphore_signal(barrier_sem, inc=1, device_id=(up,),
                                device_id_type=pl.DeviceIdType.MESH)

        @pl.when(has_dn)
        def _():
            pl.semaphore_signal(barrier_sem, inc=1, device_id=(down,),
                                device_id_type=pl.DeviceIdType.MESH)

        pl.semaphore_wait(barrier_sem, 2)

        @pl.when(has_up)
        def _():
            pl.semaphore_wait(barrier_sem, 1)

        @pl.when(has_dn)
        def _():
            pl.semaphore_wait(barrier_sem, 1)

        comm_ref[0, :, :] = x_ref[:, :].astype(jnp.bfloat16)

        sends = []

        def _rdma(src_slot, dst_slot, dev, ssem, rsem, h):
            rows = pl.ds(h * mh, mh)
            return pltpu.make_async_remote_copy(
                src_ref=comm_ref.at[src_slot, rows],
                dst_ref=comm_ref.at[dst_slot, rows],
                send_sem=ssem, recv_sem=rsem,
                device_id=(dev,), device_id_type=pl.DeviceIdType.MESH,
            )

        def _start(rdma, pred):
            if pred is None:
                rdma.start()
            else:
                @pl.when(pred)
                def _():
                    rdma.start()
            sends.append((rdma, pred))

        def _wait(rdma, pred):
            if pred is None:
                rdma.wait_recv()
            else:
                @pl.when(pred)
                def _():
                    rdma.wait_recv()


        for h in (0, 1):
            _start(_rdma(0, 1, up, csend.at[0, h], crecv.at[0, h], h),
                   has_up)
            _start(_rdma(0, 4, down, csend.at[1, h], crecv.at[3, h], h),
                   has_dn)
            _start(_rdma(0, 7, right, ssend.at[0, h], srecv.at[0, h], h),
                   None)
            _start(_rdma(0, 14, left, ssend.at[0, 2 + h],
                         srecv.at[0, 2 + h], h), None)

        w_bf[:, :] = w_ref[:, :].astype(jnp.bfloat16)
        y0 = jnp.dot(comm_ref[0, :, :], w_bf[:, :],
                     preferred_element_type=jnp.float32)
        out_ref[pl.ds(my * m_per, m_per), :] = _gelu_f32(y0)

        def _gemm_block(slot, kind, c, p):
            def _g():
                y = jnp.dot(comm_ref[slot, :, :], w_bf[:, :],
                            preferred_element_type=jnp.float32)
                out_ref[pl.ds(origin(kind, c) * m_per, m_per), :] = \
                    _gelu_f32(y)
            if p is None:
                _g()
            else:
                @pl.when(p)
                def _():
                    _g()

        def col_step(d):
            for kind in ("bel", "abv"):
                c = _cidx(kind, d)
                p = pred_c(c)
                isem = c - 1 if kind == "bel" else 3 + (c - 4)
                for h in (0, 1):
                    _wait(_rdma(c, c, left, csend.at[0, h],
                                crecv.at[isem, h], h), p)
                    if d < 3:
                        nc = c + 1
                        if kind == "bel":
                            fp = jnp.logical_and(p, has_up)
                            _start(_rdma(c, nc, up, csend.at[2 + (d - 1), h],
                                         crecv.at[nc - 1, h], h), fp)
                        else:
                            fp = jnp.logical_and(p, has_dn)
                            _start(_rdma(c, nc, down,
                                         csend.at[4 + (d - 1), h],
                                         crecv.at[3 + (nc - 4), h], h), fp)
                    _start(_rdma(c, 7 + c, right, ssend.at[c, h],
                                 srecv.at[c, h], h), p)
                    _start(_rdma(c, 14 + c, left, ssend.at[c, 2 + h],
                                 srecv.at[c, 2 + h], h), p)
                _gemm_block(c, "me", c, p)

        def sq_step(c):
            p = pred_c(c)
            _wait(_rdma(7 + c, 7 + c, left, ssend.at[c, 0],
                        srecv.at[c, 0], 0), p)
            _start(_rdma(7 + c, 21 + c, right, ssend.at[c, 4],
                         srecv.at[c, 4], 0), p)
            _wait(_rdma(14 + c, 14 + c, left, ssend.at[c, 3],
                        srecv.at[c, 3], 1), p)
            _start(_rdma(14 + c, 21 + c, left, ssend.at[c, 5],
                         srecv.at[c, 5], 1), p)
            _wait(_rdma(7 + c, 7 + c, left, ssend.at[c, 1],
                        srecv.at[c, 1], 1), p)
            _wait(_rdma(14 + c, 14 + c, left, ssend.at[c, 2],
                        srecv.at[c, 2], 0), p)
            _gemm_block(7 + c, "L", c, p)
            _gemm_block(14 + c, "R", c, p)

        def diag_step(c):
            p = pred_c(c)
            _wait(_rdma(21 + c, 21 + c, left, ssend.at[c, 4],
                        srecv.at[c, 4], 0), p)
            _wait(_rdma(21 + c, 21 + c, left, ssend.at[c, 5],
                        srecv.at[c, 5], 1), p)
            _gemm_block(21 + c, "D", c, p)

        col_step(1)
        sq_step(0)
        col_step(2)
        sq_step(1)
        sq_step(4)
        col_step(3)
        sq_step(2)
        sq_step(5)
        diag_step(0)
        sq_step(3)
        sq_step(6)
        diag_step(1)
        diag_step(4)
        diag_step(2)
        diag_step(5)
        diag_step(3)
        diag_step(6)

        for rdma, pred in sends:
            if pred is None:
                rdma.wait_send()
            else:
                @pl.when(pred)
                def _():
                    rdma.wait_send()

    return pl.pallas_call(
        body,
        out_shape=jax.ShapeDtypeStruct((N_DEV * m_per, n_per), jnp.float32),
        in_specs=[
            pl.BlockSpec(memory_space=pltpu.VMEM),
            pl.BlockSpec(memory_space=pltpu.VMEM),
        ],
        out_specs=pl.BlockSpec(memory_space=pltpu.VMEM),
        scratch_shapes=[
            pltpu.VMEM((28, m_per, k), jnp.bfloat16),
            pltpu.VMEM((k, n_per), jnp.bfloat16),
            pltpu.SemaphoreType.DMA((6, 2)),
            pltpu.SemaphoreType.DMA((6, 2)),
            pltpu.SemaphoreType.DMA((N_C, 6)),
            pltpu.SemaphoreType.DMA((N_C, 6)),
        ],
        compiler_params=pltpu.CompilerParams(collective_id=0),
    )(x, w_mat)
